# baseline (speedup 1.0000x reference)
import functools
import jax
import jax.numpy as jnp
from jax import lax
from jax.experimental import pallas as pl
from jax.experimental.pallas import tpu as pltpu
from jax.experimental.pallas import tpu_sc as plsc

N_FEAT = 1000000
N_DIM = 32
BATCH = 16384

_info = plsc.get_sparse_core_info()
_NC = _info.num_cores
_NS = _info.num_subcores
_NW = _NC * _NS
_BPW = BATCH // _NW

_mesh = plsc.VectorSubcoreMesh(core_axis_name="c", subcore_axis_name="s")


@functools.partial(
    pl.kernel,
    mesh=_mesh,
    out_type=(
        jax.ShapeDtypeStruct((BATCH,), jnp.float32),
        jax.ShapeDtypeStruct((BATCH, N_DIM), jnp.float32),
    ),
    scratch_types=[
        pltpu.VMEM((_BPW,), jnp.int32),
        pltpu.VMEM((_BPW,), jnp.float32),
        pltpu.VMEM((_BPW, N_DIM), jnp.float32),
        pltpu.SemaphoreType.DMA,
        pltpu.SemaphoreType.DMA,
    ],
    compiler_params=pltpu.CompilerParams(use_tc_tiling_on_sc=False),
)
def _lookup(idx_hbm, vect_hbm, bias_hbm, bias_out, vect_out,
            idx_v, bias_v, rows_v, sem_v, sem_b):
    wid = lax.axis_index("s") * _NC + lax.axis_index("c")
    base = wid * _BPW
    pltpu.sync_copy(idx_hbm.at[pl.ds(base, _BPW)], idx_v)
    cv = pltpu.async_copy(vect_hbm.at[idx_v], rows_v, sem_v)
    cb = pltpu.async_copy(bias_hbm.at[idx_v], bias_v, sem_b)
    cv.wait()
    cb.wait()
    pltpu.sync_copy(rows_v, vect_out.at[pl.ds(base, _BPW)])
    pltpu.sync_copy(bias_v, bias_out.at[pl.ds(base, _BPW)])


def kernel(index, vect, bias):
    idx = index.astype(jnp.int32)
    bias_out, vect_out = _lookup(idx, vect, bias.reshape(N_FEAT))
    return bias_out.reshape(BATCH, 1), vect_out
